# baseline (speedup 1.0000x reference)
import functools

import jax
import jax.numpy as jnp
from jax.experimental import pallas as pl


def _onehot_body(row_ref, out_ref, *, n_blk, n_classes, n_tok):
    j = pl.program_id(1)
    n_j = pl.num_programs(1)
    blk = row_ref[0, :, pl.ds(j * n_blk, n_blk)]
    wrap = jax.lax.rem(j + 1, n_j)
    nxt_chunk = row_ref[0, :, pl.ds(wrap * n_blk, 128)]
    nxt = jnp.where(j == n_j - 1, blk[:, -1:] + 1, nxt_chunk[:, :1])
    shifted = jnp.concatenate([blk[:, 1:], nxt], axis=1)
    shifted = jnp.where(shifted > n_classes - 1, 0, shifted)
    col = jnp.transpose(shifted, (1, 0))
    iota = jax.lax.broadcasted_iota(jnp.int32, (n_blk, n_classes), 1)
    out_ref[0] = jnp.where(col == iota, jnp.float32(1.0), jnp.float32(0.0))


def kernel(inputs):
    B, T = inputs.shape
    K = 256
    N = 1024
    rows = inputs.astype(jnp.int32).reshape(B, 1, T)
    return pl.pallas_call(
        functools.partial(_onehot_body, n_blk=N, n_classes=K, n_tok=T),
        grid=(B, T // N),
        in_specs=[pl.BlockSpec((1, 1, T), lambda i, j: (i, 0, 0))],
        out_specs=pl.BlockSpec((1, N, K), lambda i, j: (i, j, 0)),
        out_shape=jax.ShapeDtypeStruct((B, T, K), jnp.float32),
    )(rows)


# speedup vs baseline: 1.7752x; 1.7752x over previous
import functools

import jax
import jax.numpy as jnp
from jax.experimental import pallas as pl
from jax.experimental.pallas import tpu as pltpu


def _onehot_body(row_ref, out_ref, *, n_classes):
    row = row_ref[0]
    shifted = jnp.concatenate([row[:, 1:], row[:, -1:] + 1], axis=1)
    shifted = jnp.where(shifted > n_classes - 1, 0, shifted)
    col = jnp.transpose(shifted, (1, 0))
    iota = jax.lax.broadcasted_iota(jnp.int32, out_ref.shape[1:], 1)
    out_ref[0] = jnp.where(col == iota, jnp.float32(1.0), jnp.float32(0.0))


def kernel(inputs):
    B, T = inputs.shape
    K = 256
    rows = inputs.astype(jnp.int32).reshape(B, 1, T)
    return pl.pallas_call(
        functools.partial(_onehot_body, n_classes=K),
        grid=(B,),
        in_specs=[pl.BlockSpec((1, 1, T), lambda i: (i, 0, 0))],
        out_specs=pl.BlockSpec((1, T, K), lambda i: (i, 0, 0)),
        out_shape=jax.ShapeDtypeStruct((B, T, K), jnp.float32),
        compiler_params=pltpu.CompilerParams(
            dimension_semantics=("parallel",),
        ),
    )(rows)


# speedup vs baseline: 1.9668x; 1.1079x over previous
import functools

import jax
import jax.numpy as jnp
from jax.experimental import pallas as pl
from jax.experimental.pallas import tpu as pltpu


def _onehot_body(cols_ref, nxt_ref, out_ref, *, n_blk, n_classes, n_batch):
    j = pl.program_id(0)
    n_j = pl.num_programs(0)
    blk = cols_ref[...]
    nxt = jnp.where(j == n_j - 1, blk[-1:, :] + 1, nxt_ref[0:1, :])
    shifted = jnp.concatenate([blk[1:, :], nxt], axis=0)
    shifted = jnp.where(shifted > n_classes - 1, 0, shifted)
    iota = jax.lax.broadcasted_iota(jnp.int32, (n_blk, n_classes), 1)
    for b in range(n_batch):
        col = shifted[:, b:b + 1]
        out_ref[b] = jnp.where(col == iota, jnp.float32(1.0), jnp.float32(0.0))


def kernel(inputs):
    B, T = inputs.shape
    K = 256
    N = 128
    C = T // N
    cols = jnp.transpose(inputs.astype(jnp.int32))
    return pl.pallas_call(
        functools.partial(_onehot_body, n_blk=N, n_classes=K, n_batch=B),
        grid=(C,),
        in_specs=[
            pl.BlockSpec((N, B), lambda j: (j, 0)),
            pl.BlockSpec((8, B), lambda j, c=C, w=N // 8: (((j + 1) % c) * w, 0)),
        ],
        out_specs=pl.BlockSpec((B, N, K), lambda j: (0, j, 0)),
        out_shape=jax.ShapeDtypeStruct((B, T, K), jnp.float32),
        compiler_params=pltpu.CompilerParams(
            dimension_semantics=("arbitrary",),
        ),
    )(cols, cols)


# speedup vs baseline: 2.0801x; 1.0576x over previous
import functools

import jax
import jax.numpy as jnp
from jax.experimental import pallas as pl
from jax.experimental.pallas import tpu as pltpu


def _onehot_body(rows_ref, out_ref, cols_ref, *, n_blk, n_classes, n_batch):
    j = pl.program_id(0)

    @pl.when(j == 0)
    def _prep():
        cols = jnp.transpose(rows_ref[...], (1, 0))
        shifted = jnp.concatenate([cols[1:, :], cols[-1:, :] + 1], axis=0)
        cols_ref[...] = jnp.where(shifted > n_classes - 1, 0, shifted)

    blk = cols_ref[pl.ds(j * n_blk, n_blk), :]
    iota = jax.lax.broadcasted_iota(jnp.int32, (n_blk, n_classes), 1)
    for b in range(n_batch):
        col = blk[:, b:b + 1]
        out_ref[b] = jnp.where(col == iota, jnp.float32(1.0), jnp.float32(0.0))


def kernel(inputs):
    B, T = inputs.shape
    K = 256
    N = 128
    C = T // N
    return pl.pallas_call(
        functools.partial(_onehot_body, n_blk=N, n_classes=K, n_batch=B),
        grid=(C,),
        in_specs=[pl.BlockSpec((B, T), lambda j: (0, 0))],
        out_specs=pl.BlockSpec((B, N, K), lambda j: (0, j, 0)),
        out_shape=jax.ShapeDtypeStruct((B, T, K), jnp.float32),
        scratch_shapes=[pltpu.VMEM((T, B), jnp.int32)],
        compiler_params=pltpu.CompilerParams(
            dimension_semantics=("arbitrary",),
        ),
    )(inputs.astype(jnp.int32))
